# baseline (speedup 1.0000x reference)
import functools

import jax
import jax.numpy as jnp
from jax import lax
from jax.experimental import pallas as pl
from jax.experimental.pallas import tpu as pltpu
from jax.experimental.pallas import tpu_sc as plsc

NC = 2
NS = 16
NW = NC * NS
LANES = 16

BCOL = 128
SBLK = 8


@jax.jit
def _fused_embed(xT, tpad, peb):
    mesh = plsc.VectorSubcoreMesh(core_axis_name="c", subcore_axis_name="s")

    @functools.partial(
        pl.kernel,
        out_type=jax.ShapeDtypeStruct((200, 64, 4096), jnp.float32),
        mesh=mesh,
        scratch_types=[
            pltpu.VMEM((SBLK, BCOL), jnp.int32),
            pltpu.VMEM((SBLK, 64 * LANES), jnp.float32),
            pltpu.VMEM((BCOL, 128), jnp.float32),
            pltpu.VMEM((64, BCOL), jnp.float32),
            pltpu.SemaphoreType.DMA,
        ],
        compiler_params=pltpu.CompilerParams(needs_layout_passes=False),
    )
    def body(xT_hbm, tpad_hbm, peb_hbm, out_hbm, idx_v, peb_v, rows_v, tout_v, sem):
        wid = lax.axis_index("s") * NC + lax.axis_index("c")
        bcol = wid * BCOL
        row_iota = lax.iota(jnp.int32, LANES)

        def sblk_body(sb, carry):
            s0 = sb * SBLK
            pltpu.sync_copy(xT_hbm.at[pl.ds(s0, SBLK), pl.ds(bcol, BCOL)], idx_v)
            pltpu.sync_copy(peb_hbm.at[pl.ds(s0, SBLK)], peb_v)

            def s_body(sl, scarry):
                pltpu.async_copy(tpad_hbm.at[idx_v.at[sl]], rows_v, sem).wait()
                for d in range(64):
                    pev = peb_v[sl, pl.ds(d * LANES, LANES)]
                    cidx = jnp.full((LANES,), d, jnp.int32)
                    for j in range(BCOL // LANES):
                        ridx = row_iota + (j * LANES)
                        vals = plsc.load_gather(rows_v, [ridx, cidx])
                        tout_v[d, pl.ds(j * LANES, LANES)] = vals + pev
                pltpu.sync_copy(
                    tout_v, out_hbm.at[s0 + sl, slice(None), pl.ds(bcol, BCOL)]
                )
                return scarry

            lax.fori_loop(0, SBLK, s_body, 0)
            return carry

        lax.fori_loop(0, 200 // SBLK, sblk_body, 0)

    return body(xT, tpad, peb)


def kernel(x, table, pe):
    batch, seq = x.shape
    emb_dim = table.shape[1]
    xT = x.T
    tpad = jnp.pad(table, ((0, 0), (0, 128 - emb_dim)))
    peb = jnp.repeat(pe[:seq], LANES, axis=1)
    outp = _fused_embed(xT, tpad, peb)
    return jnp.transpose(outp, (2, 0, 1))


# speedup vs baseline: 1.1000x; 1.1000x over previous
import functools

import jax
import jax.numpy as jnp
from jax import lax
from jax.experimental import pallas as pl
from jax.experimental.pallas import tpu as pltpu
from jax.experimental.pallas import tpu_sc as plsc

NC = 2
NS = 16
NW = NC * NS
LANES = 16

SEQ = 200
EMB = 64
BCOL = 128


@jax.jit
def _fused_embed(xT, tpad, pe200):
    mesh = plsc.VectorSubcoreMesh(core_axis_name="c", subcore_axis_name="s")

    @functools.partial(
        pl.kernel,
        out_type=jax.ShapeDtypeStruct((SEQ, EMB, NW * BCOL), jnp.float32),
        mesh=mesh,
        scratch_types=[
            pltpu.VMEM((SEQ, BCOL), jnp.int32),
            pltpu.VMEM((SEQ, EMB), jnp.float32),
            pltpu.VMEM((2, BCOL, 128), jnp.float32),
            pltpu.VMEM((2, EMB, BCOL), jnp.float32),
            pltpu.SemaphoreType.DMA,
            pltpu.SemaphoreType.DMA,
            pltpu.SemaphoreType.DMA,
            pltpu.SemaphoreType.DMA,
        ],
        compiler_params=pltpu.CompilerParams(needs_layout_passes=False),
    )
    def body(xT_hbm, tpad_hbm, pe_hbm, out_hbm,
             idx_v, pe_v, rows_v, tout_v, gsem0, gsem1, wsem0, wsem1):
        wid = lax.axis_index("s") * NC + lax.axis_index("c")
        bcol = wid * BCOL
        iota = lax.iota(jnp.int32, LANES)

        pltpu.sync_copy(xT_hbm.at[slice(None), pl.ds(bcol, BCOL)], idx_v)
        pltpu.sync_copy(pe_hbm, pe_v)

        gsems = (gsem0, gsem1)
        wsems = (wsem0, wsem1)
        pltpu.async_copy(tpad_hbm.at[idx_v.at[0]], rows_v.at[0], gsem0)
        pltpu.async_copy(tpad_hbm.at[idx_v.at[1]], rows_v.at[1], gsem1)

        def step(i, carry):
            for p in range(2):
                s = i * 2 + p
                rv, tv = rows_v.at[p], tout_v.at[p]
                gs, ws = gsems[p], wsems[p]
                pltpu.make_async_copy(tpad_hbm.at[idx_v.at[s]], rv, gs).wait()

                @pl.when(i > 0)
                def _():
                    pltpu.make_async_copy(
                        tv, out_hbm.at[s - 2, slice(None), pl.ds(bcol, BCOL)],
                        ws).wait()

                spl = jnp.full((LANES,), s, jnp.int32)
                for d in range(EMB):
                    cidx = jnp.full((LANES,), d, jnp.int32)
                    pev = plsc.load_gather(pe_v, [spl, cidx])
                    for j in range(BCOL // LANES):
                        vals = plsc.load_gather(rv, [iota + (j * LANES), cidx])
                        tv[d, pl.ds(j * LANES, LANES)] = vals + pev

                @pl.when(s + 2 < SEQ)
                def _():
                    pltpu.async_copy(tpad_hbm.at[idx_v.at[s + 2]], rv, gs)

                pltpu.async_copy(
                    tv, out_hbm.at[s, slice(None), pl.ds(bcol, BCOL)], ws)
            return carry

        lax.fori_loop(0, SEQ // 2, step, 0)
        for p in range(2):
            pltpu.make_async_copy(
                tout_v.at[p],
                out_hbm.at[SEQ - 2 + p, slice(None), pl.ds(bcol, BCOL)],
                wsems[p]).wait()

    return body(xT, tpad, pe200)


def kernel(x, table, pe):
    batch, seq = x.shape
    emb_dim = table.shape[1]
    xT = x.T
    tpad = jnp.pad(table, ((0, 0), (0, 128 - emb_dim)))
    outp = _fused_embed(xT, tpad, pe[:seq])
    return jnp.transpose(outp, (2, 0, 1))


# speedup vs baseline: 1.8566x; 1.6878x over previous
import functools

import jax
import jax.numpy as jnp
from jax import lax
from jax.experimental import pallas as pl
from jax.experimental.pallas import tpu as pltpu
from jax.experimental.pallas import tpu_sc as plsc

NC = 2
NS = 16
NW = NC * NS
LANES = 16

SEQ = 200
EMB = 64
BCOL = 128


@jax.jit
def _fused_embed(xT, tpad, pe200):
    mesh = plsc.VectorSubcoreMesh(core_axis_name="c", subcore_axis_name="s")

    @functools.partial(
        pl.kernel,
        out_type=jax.ShapeDtypeStruct((SEQ, EMB, NW * BCOL), jnp.float32),
        mesh=mesh,
        scratch_types=[
            pltpu.VMEM((SEQ, BCOL), jnp.int32),
            pltpu.VMEM((SEQ, EMB), jnp.float32),
            pltpu.VMEM((2, BCOL, 128), jnp.float32),
            pltpu.VMEM((2, EMB, BCOL), jnp.float32),
            pltpu.SemaphoreType.DMA,
            pltpu.SemaphoreType.DMA,
            pltpu.SemaphoreType.DMA,
            pltpu.SemaphoreType.DMA,
        ],
        compiler_params=pltpu.CompilerParams(needs_layout_passes=False),
    )
    def body(xT_hbm, tpad_hbm, pe_hbm, out_hbm,
             idx_v, pe_v, rows_v, tout_v, gsem0, gsem1, wsem0, wsem1):
        wid = lax.axis_index("s") * NC + lax.axis_index("c")
        bcol = wid * BCOL
        iota = lax.iota(jnp.int32, LANES)

        pltpu.sync_copy(xT_hbm.at[slice(None), pl.ds(bcol, BCOL)], idx_v)
        pltpu.sync_copy(pe_hbm, pe_v)

        gsems = (gsem0, gsem1)
        wsems = (wsem0, wsem1)
        pltpu.async_copy(tpad_hbm.at[idx_v.at[0]], rows_v.at[0], gsem0)
        pltpu.async_copy(tpad_hbm.at[idx_v.at[1]], rows_v.at[1], gsem1)

        def step(i, carry):
            for p in range(2):
                s = i * 2 + p
                rv, tv = rows_v.at[p], tout_v.at[p]
                gs, ws = gsems[p], wsems[p]
                pltpu.make_async_copy(tpad_hbm.at[idx_v.at[s]], rv, gs).wait()

                @pl.when(i > 0)
                def _():
                    pltpu.make_async_copy(
                        tv, out_hbm.at[s - 2, slice(None), pl.ds(bcol, BCOL)],
                        ws).wait()

                spl = jnp.full((LANES,), s, jnp.int32)

                @plsc.parallel_loop(0, EMB, unroll=8)
                def _(d):
                    cidx = jnp.full((LANES,), d, jnp.int32)
                    pev = plsc.load_gather(pe_v, [spl, cidx])
                    for j in range(BCOL // LANES):
                        vals = plsc.load_gather(rv, [iota + (j * LANES), cidx])
                        tv[d, pl.ds(j * LANES, LANES)] = vals + pev

                @pl.when(s + 2 < SEQ)
                def _():
                    pltpu.async_copy(tpad_hbm.at[idx_v.at[s + 2]], rv, gs)

                pltpu.async_copy(
                    tv, out_hbm.at[s, slice(None), pl.ds(bcol, BCOL)], ws)
            return carry

        lax.fori_loop(0, SEQ // 2, step, 0)
        for p in range(2):
            pltpu.make_async_copy(
                tout_v.at[p],
                out_hbm.at[SEQ - 2 + p, slice(None), pl.ds(bcol, BCOL)],
                wsems[p]).wait()

    return body(xT, tpad, pe200)


def kernel(x, table, pe):
    batch, seq = x.shape
    emb_dim = table.shape[1]
    xT = x.T
    tpad = jnp.pad(table, ((0, 0), (0, 128 - emb_dim)))
    outp = _fused_embed(xT, tpad, pe[:seq])
    return jnp.transpose(outp, (2, 0, 1))
